# baseline (speedup 1.0000x reference)
import functools

import jax
import jax.numpy as jnp
from jax import lax
from jax.experimental import pallas as pl
from jax.experimental.pallas import tpu as pltpu
from jax.experimental.pallas import tpu_sc as plsc

_B = 256
_NL = 12
_KD = 768
_NT = 10
_NP = 8
_ED = 768
_NK = 100
_ROWS = _NL * _B
_NC = 2
_NS = 16
_NW = _NC * _NS
_RPW = _ROWS // _NW
_CH = 8
_NCHUNK = _RPW // _CH


def _match_body(q_ref, keys_ref, out_ref):
    q = q_ref[0]
    keys = keys_ref[...]
    knorm = jax.lax.dot_general(
        keys * keys, jnp.ones((1, _KD), jnp.float32),
        (((1,), (1,)), ((), ())), preferred_element_type=jnp.float32)
    cross = jax.lax.dot_general(
        keys, q, (((1,), (1,)), ((), ())),
        preferred_element_type=jnp.float32)
    scores = knorm - 2.0 * cross
    rowidx = jax.lax.broadcasted_iota(jnp.int32, (_NK, _B), 0)
    mval = jnp.min(scores, axis=0, keepdims=True)
    idx = jnp.min(jnp.where(scores == mval, rowidx, _NK), axis=0, keepdims=True)
    task = idx // _NK
    out_ref[0] = task + _NT * pl.program_id(0)


def _match_rows(xq, task_keys):
    return pl.pallas_call(
        _match_body,
        grid=(_NL,),
        in_specs=[
            pl.BlockSpec((1, _B, _KD), lambda l: (l, 0, 0)),
            pl.BlockSpec((_NK, _KD), lambda l: (0, 0)),
        ],
        out_specs=pl.BlockSpec((1, 1, _B), lambda l: (l, 0, 0)),
        out_shape=jax.ShapeDtypeStruct((_NL, 1, _B), jnp.int32),
    )(xq, task_keys)


def _sc_gather(table, rows):
    mesh = plsc.VectorSubcoreMesh(core_axis_name="c", subcore_axis_name="s")

    @functools.partial(
        pl.kernel, mesh=mesh,
        out_type=jax.ShapeDtypeStruct((_NL, _B, _NP, _ED), jnp.float32),
        scratch_types=[
            pltpu.VMEM((_RPW,), jnp.int32),
            pltpu.VMEM((2, _CH, _NP, _ED), jnp.float32),
            pltpu.SemaphoreType.DMA,
            pltpu.SemaphoreType.DMA,
            pltpu.SemaphoreType.DMA,
            pltpu.SemaphoreType.DMA,
        ],
    )
    def gather(table_hbm, idx_hbm, out_hbm, idx_v, buf_v, g0, g1, w0, w1):
        wid = lax.axis_index("s") * _NC + lax.axis_index("c")
        base = wid * _RPW
        pltpu.sync_copy(idx_hbm.at[pl.ds(base, _RPW)], idx_v)
        gsem = (g0, g1)
        wsem = (w0, w1)

        def gstart(c, s):
            return pltpu.async_copy(
                table_hbm.at[idx_v.at[pl.ds(c * _CH, _CH)]],
                buf_v.at[s], gsem[s])

        def wstart(c, s):
            r = base + c * _CH
            return pltpu.async_copy(
                buf_v.at[s], out_hbm.at[r // _B, pl.ds(r % _B, _CH)], wsem[s])

        gd = [gstart(0, 0), gstart(1, 1)]
        wd = [None, None]
        for c in range(_NCHUNK):
            s = c & 1
            gd[s].wait()
            wd[s] = wstart(c, s)
            if c + 2 < _NCHUNK:
                wd[s].wait()
                gd[s] = gstart(c + 2, s)
        wd[0].wait()
        wd[1].wait()

    return gather(table, rows)


def kernel(x_query, vis_mark, P, task_keys):
    del vis_mark
    xq = jnp.transpose(x_query, (1, 0, 2))
    rows = _match_rows(xq, task_keys).reshape(_ROWS)
    table = P.reshape(_NL * _NT, _NP, _ED)
    out = _sc_gather(table, rows)
    return (out, jnp.float32(0.0))


# speedup vs baseline: 2.5588x; 2.5588x over previous
import jax
import jax.numpy as jnp
from jax.experimental import pallas as pl
from jax.experimental.pallas import tpu as pltpu

_B = 256
_NL = 12
_KD = 768
_NT = 10
_NP = 8
_ED = 768
_NK = 100
_BP = _B * _NP
_TP = _NT * _NP


def _layer_body(q_ref, keys_ref, p_ref, out_ref, e8_ref, cdiv_ref, mask_ref):
    @pl.when(pl.program_id(0) == 0)
    def _init():
        r = jax.lax.broadcasted_iota(jnp.int32, (_BP, _B), 0)
        b = jax.lax.broadcasted_iota(jnp.int32, (_BP, _B), 1)
        e8_ref[...] = ((r // _NP) == b).astype(jnp.float32)
        rp = jax.lax.broadcasted_iota(jnp.int32, (_BP, _TP), 0)
        cp = jax.lax.broadcasted_iota(jnp.int32, (_BP, _TP), 1)
        cdiv_ref[...] = cp // _NP
        mask_ref[...] = ((rp % _NP) == (cp % _NP)).astype(jnp.float32)

    q = q_ref[:, pl.program_id(0), :]
    keys = keys_ref[...]
    knorm = jax.lax.dot_general(
        jnp.ones((1, _KD), jnp.float32), keys * keys,
        (((1,), (1,)), ((), ())), preferred_element_type=jnp.float32)
    cross = jax.lax.dot_general(
        q, keys, (((1,), (1,)), ((), ())),
        preferred_element_type=jnp.float32)
    scores = knorm - 2.0 * cross
    colidx = jax.lax.broadcasted_iota(jnp.int32, (_B, _NK), 1)
    mval = jnp.min(scores, axis=1, keepdims=True)
    idx = jnp.min(jnp.where(scores == mval, colidx, _NK), axis=1, keepdims=True)
    taskf = (idx // _NK).astype(jnp.float32)
    t2048 = jax.lax.dot_general(
        e8_ref[...], taskf, (((1,), (0,)), ((), ())),
        preferred_element_type=jnp.float32)
    onehot2 = jnp.where(t2048.astype(jnp.int32) == cdiv_ref[...],
                        mask_ref[...], 0.0)
    res = jax.lax.dot_general(
        onehot2, p_ref[0], (((1,), (0,)), ((), ())),
        preferred_element_type=jnp.float32)
    out_ref[0] = res.reshape(_B, _NP, _ED)


def kernel(x_query, vis_mark, P, task_keys):
    del vis_mark
    p2 = P.reshape(_NL, _TP, _ED)
    out = pl.pallas_call(
        _layer_body,
        grid=(_NL,),
        in_specs=[
            pl.BlockSpec((_B, _NL, _KD), lambda l: (0, 0, 0)),
            pl.BlockSpec((_NK, _KD), lambda l: (0, 0)),
            pl.BlockSpec((1, _TP, _ED), lambda l: (l, 0, 0)),
        ],
        out_specs=pl.BlockSpec((1, _B, _NP, _ED), lambda l: (l, 0, 0, 0)),
        out_shape=jax.ShapeDtypeStruct((_NL, _B, _NP, _ED), jnp.float32),
        scratch_shapes=[
            pltpu.VMEM((_BP, _B), jnp.float32),
            pltpu.VMEM((_BP, _TP), jnp.int32),
            pltpu.VMEM((_BP, _TP), jnp.float32),
        ],
    )(x_query, task_keys, p2)
    return (out, jnp.float32(0.0))
